# Initial kernel scaffold; baseline (speedup 1.0000x reference)
import jax
import jax.numpy as jnp
from jax.experimental import pallas as pl


def kernel(data, centers):
    raise NotImplementedError("write your pallas kernel here")


# speedup vs baseline: 14.8416x; 14.8416x over previous
import math

import jax
import jax.numpy as jnp
from jax.experimental import pallas as pl

_NUM = 256
_N = 16384
_D = 5
_MAX_ITERS = 10
_THRESHOLD = 0.0001
_R = 0.005 * math.sqrt(_N / _NUM)
_ROWS = 128
_COLS = 128


def _cluster_body(data_ref, ct_ref, lab_ref, cout_ref):
    X = [data_ref[d] for d in range(_D)]
    lanes = jax.lax.broadcasted_iota(jnp.int32, (1, _NUM), 1)
    zero_row = jnp.zeros((1, _NUM), jnp.float32)

    def pick(row, i):
        return jnp.sum(jnp.where(lanes == i, row, 0.0))

    def make_assign_body(Ct):
        def body(i, carry):
            lab, dv = carry
            c = [pick(Ct[d:d + 1, :], i) for d in range(_D)]
            sp = jnp.sqrt((X[0] - c[0]) ** 2 + (X[1] - c[1]) ** 2)
            mask = sp <= _R
            countf = jnp.sum(mask.astype(jnp.float32))
            dist2 = ((X[0] - c[0] + 1e-6) ** 2
                     + (X[1] - c[1] + 1e-6) ** 2
                     + (X[2] - c[2] + 1e-6) ** 2
                     + (X[3] - c[3] + 1e-6) ** 2
                     + (X[4] - c[4] + 1e-6) ** 2)
            dist = jnp.sqrt(dist2)
            upd = mask & (dist < dv)
            do = mask & (countf > 1.5)
            dv = jnp.where(do, jnp.where(upd, dist, 0.0), dv)
            lab = jnp.where(do, jnp.where(upd, i, lab), lab)
            return lab, dv
        return body

    def make_seg_body(lab):
        def body(i, carry):
            cnt_row = carry[0]
            accs = carry[1:]
            onehot = jnp.where(lanes == i, 1.0, 0.0)
            eq = lab == i
            cnt = jnp.sum(eq.astype(jnp.float32))
            new_accs = tuple(
                accs[d] + jnp.sum(jnp.where(eq, X[d], 0.0)) * onehot
                for d in range(_D))
            return (cnt_row + cnt * onehot,) + new_accs
        return body

    def step(carry):
        lab, dv, Ct, done = carry
        lab, dv = jax.lax.fori_loop(0, _NUM, make_assign_body(Ct), (lab, dv))
        seg = jax.lax.fori_loop(0, _NUM, make_seg_body(lab),
                                (zero_row,) * (_D + 1))
        cnt_row = seg[0]
        empty = cnt_row == 0.0
        denom = jnp.where(empty, 1.0, cnt_row)
        rows = [jnp.where(empty, Ct[d:d + 1, :], seg[1 + d] / denom)
                for d in range(_D)]
        new_ct = jnp.concatenate(rows + [zero_row] * 3, axis=0)
        diff = new_ct - Ct
        nrm = jnp.sqrt(jnp.sum(diff * diff))
        return lab, dv, new_ct, done | (nrm < _THRESHOLD)

    def skip(carry):
        lab, dv, Ct, done = carry
        return lab, dv, Ct, jnp.bool_(True)

    def outer(_, carry):
        return jax.lax.cond(carry[3], skip, step, carry)

    lab0 = jnp.full((_ROWS, _COLS), -1, jnp.int32)
    dv0 = jnp.full((_ROWS, _COLS), 1000.0, jnp.float32)
    lab, dv, ct, done = jax.lax.fori_loop(
        0, _MAX_ITERS, outer, (lab0, dv0, ct_ref[...], jnp.bool_(False)))
    lab_ref[...] = lab
    cout_ref[...] = ct


def kernel(data, centers):
    data_r = data.T.reshape(_D, _ROWS, _COLS)
    ct_t = jnp.zeros((8, _NUM), jnp.float32).at[:_D].set(centers.T)
    lab, cout = pl.pallas_call(
        _cluster_body,
        out_shape=[
            jax.ShapeDtypeStruct((_ROWS, _COLS), jnp.int32),
            jax.ShapeDtypeStruct((8, _NUM), jnp.float32),
        ],
    )(data_r, ct_t)
    return lab.reshape(_N), cout[:_D].T


# speedup vs baseline: 24.8723x; 1.6759x over previous
import math

import jax
import jax.numpy as jnp
from jax.experimental import pallas as pl

_NUM = 256
_N = 16384
_D = 5
_MAX_ITERS = 10
_THRESHOLD = 0.0001
_R = 0.005 * math.sqrt(_N / _NUM)
_ROWS = 128
_COLS = 128


def _cluster_body(data_ref, ct_ref, lab_ref, cout_ref):
    X = [data_ref[d] for d in range(_D)]
    lanes = jax.lax.broadcasted_iota(jnp.int32, (1, _NUM), 1)
    zero_row = jnp.zeros((1, _NUM), jnp.float32)

    def pick(row, i):
        return jnp.sum(jnp.where(lanes == i, row, 0.0))

    def make_assign_body(Ct):
        def body(i, carry):
            lab, dv = carry
            c = [pick(Ct[d:d + 1, :], i) for d in range(_D)]
            sp = jnp.sqrt((X[0] - c[0]) ** 2 + (X[1] - c[1]) ** 2)
            mask = sp <= _R
            countf = jnp.sum(mask.astype(jnp.float32))
            dist2 = ((X[0] - c[0] + 1e-6) ** 2
                     + (X[1] - c[1] + 1e-6) ** 2
                     + (X[2] - c[2] + 1e-6) ** 2
                     + (X[3] - c[3] + 1e-6) ** 2
                     + (X[4] - c[4] + 1e-6) ** 2)
            dist = jnp.sqrt(dist2)
            upd = mask & (dist < dv)
            do = mask & (countf > 1.5)
            dv = jnp.where(do, jnp.where(upd, dist, 0.0), dv)
            lab = jnp.where(do, jnp.where(upd, i, lab), lab)
            return lab, dv
        return body

    def make_seg_body(lab):
        def body(i, carry):
            cnt_row = carry[0]
            accs = carry[1:]
            onehot = jnp.where(lanes == i, 1.0, 0.0)
            eq = lab == i
            cnt = jnp.sum(eq.astype(jnp.float32))
            new_accs = tuple(
                accs[d] + jnp.sum(jnp.where(eq, X[d], 0.0)) * onehot
                for d in range(_D))
            return (cnt_row + cnt * onehot,) + new_accs
        return body

    def step(carry):
        lab, dv, Ct, done = carry
        lab, dv = jax.lax.fori_loop(0, _NUM, make_assign_body(Ct), (lab, dv),
                                    unroll=8)
        seg = jax.lax.fori_loop(0, _NUM, make_seg_body(lab),
                                (zero_row,) * (_D + 1), unroll=8)
        cnt_row = seg[0]
        empty = cnt_row == 0.0
        denom = jnp.where(empty, 1.0, cnt_row)
        rows = [jnp.where(empty, Ct[d:d + 1, :], seg[1 + d] / denom)
                for d in range(_D)]
        new_ct = jnp.concatenate(rows + [zero_row] * 3, axis=0)
        diff = new_ct - Ct
        nrm = jnp.sqrt(jnp.sum(diff * diff))
        return lab, dv, new_ct, done | (nrm < _THRESHOLD)

    def skip(carry):
        lab, dv, Ct, done = carry
        return lab, dv, Ct, jnp.bool_(True)

    def outer(_, carry):
        return jax.lax.cond(carry[3], skip, step, carry)

    lab0 = jnp.full((_ROWS, _COLS), -1, jnp.int32)
    dv0 = jnp.full((_ROWS, _COLS), 1000.0, jnp.float32)
    lab, dv, ct, done = jax.lax.fori_loop(
        0, _MAX_ITERS, outer, (lab0, dv0, ct_ref[...], jnp.bool_(False)))
    lab_ref[...] = lab
    cout_ref[...] = ct


def kernel(data, centers):
    data_r = data.T.reshape(_D, _ROWS, _COLS)
    ct_t = jnp.zeros((8, _NUM), jnp.float32).at[:_D].set(centers.T)
    lab, cout = pl.pallas_call(
        _cluster_body,
        out_shape=[
            jax.ShapeDtypeStruct((_ROWS, _COLS), jnp.int32),
            jax.ShapeDtypeStruct((8, _NUM), jnp.float32),
        ],
    )(data_r, ct_t)
    return lab.reshape(_N), cout[:_D].T


# speedup vs baseline: 39.3891x; 1.5837x over previous
import math

import jax
import jax.numpy as jnp
from jax.experimental import pallas as pl

_NUM = 256
_N = 16384
_D = 5
_MAX_ITERS = 10
_THRESHOLD = 0.0001
_R = 0.005 * math.sqrt(_N / _NUM)
_ROWS = 128
_COLS = 128


def _cluster_body(data_ref, ct_ref, lab_ref, cout_ref):
    X = [data_ref[d] for d in range(_D)]
    lanes = jax.lax.broadcasted_iota(jnp.int32, (1, _NUM), 1)
    zero_row = jnp.zeros((1, _NUM), jnp.float32)

    def pick(row, i):
        return jnp.sum(jnp.where(lanes == i, row, 0.0))

    def make_assign_body(Ct):
        def body(i, carry):
            lab, dv = carry
            c_col = jnp.sum(jnp.where(lanes == i, Ct, 0.0), axis=1,
                            keepdims=True)
            t = [X[d] - c_col[d:d + 1, 0:1] for d in range(_D)]
            sp = jnp.sqrt(t[0] * t[0] + t[1] * t[1])
            mask = sp <= _R
            countf = jnp.sum(mask.astype(jnp.float32))
            e = [(t[d] + 1e-6) for d in range(_D)]
            dist2 = (e[0] * e[0] + e[1] * e[1] + e[2] * e[2]
                     + e[3] * e[3] + e[4] * e[4])
            dist = jnp.sqrt(dist2)
            upd = mask & (dist < dv)
            do = mask & (countf > 1.5)
            dv = jnp.where(do, jnp.where(upd, dist, 0.0), dv)
            lab = jnp.where(do, jnp.where(upd, i, lab), lab)
            return lab, dv
        return body

    def make_seg_body(lab):
        def body(i, carry):
            cnt_row = carry[0]
            accs = carry[1:]
            onehot = jnp.where(lanes == i, 1.0, 0.0)
            eq = lab == i
            cnt = jnp.sum(eq.astype(jnp.float32))
            new_accs = tuple(
                accs[d] + jnp.sum(jnp.where(eq, X[d], 0.0)) * onehot
                for d in range(_D))
            return (cnt_row + cnt * onehot,) + new_accs
        return body

    def step(carry):
        lab, dv, Ct, done = carry
        lab, dv = jax.lax.fori_loop(0, _NUM, make_assign_body(Ct), (lab, dv),
                                    unroll=16)
        seg = jax.lax.fori_loop(0, _NUM, make_seg_body(lab),
                                (zero_row,) * (_D + 1), unroll=16)
        cnt_row = seg[0]
        empty = cnt_row == 0.0
        denom = jnp.where(empty, 1.0, cnt_row)
        rows = [jnp.where(empty, Ct[d:d + 1, :], seg[1 + d] / denom)
                for d in range(_D)]
        new_ct = jnp.concatenate(rows + [zero_row] * 3, axis=0)
        diff = new_ct - Ct
        nrm = jnp.sqrt(jnp.sum(diff * diff))
        return lab, dv, new_ct, done | (nrm < _THRESHOLD)

    def skip(carry):
        lab, dv, Ct, done = carry
        return lab, dv, Ct, jnp.bool_(True)

    def outer(_, carry):
        return jax.lax.cond(carry[3], skip, step, carry)

    lab0 = jnp.full((_ROWS, _COLS), -1, jnp.int32)
    dv0 = jnp.full((_ROWS, _COLS), 1000.0, jnp.float32)
    lab, dv, ct, done = jax.lax.fori_loop(
        0, _MAX_ITERS, outer, (lab0, dv0, ct_ref[...], jnp.bool_(False)))
    lab_ref[...] = lab
    cout_ref[...] = ct


def kernel(data, centers):
    data_r = data.T.reshape(_D, _ROWS, _COLS)
    ct_t = jnp.zeros((8, _NUM), jnp.float32).at[:_D].set(centers.T)
    lab, cout = pl.pallas_call(
        _cluster_body,
        out_shape=[
            jax.ShapeDtypeStruct((_ROWS, _COLS), jnp.int32),
            jax.ShapeDtypeStruct((8, _NUM), jnp.float32),
        ],
    )(data_r, ct_t)
    return lab.reshape(_N), cout[:_D].T
